# baseline (speedup 1.0000x reference)
import functools
import math

import jax
import jax.numpy as jnp
from jax import lax
from jax.experimental import pallas as pl
from jax.experimental.pallas import tpu as pltpu
from jax.experimental.pallas import tpu_sc as plsc

N = 50000
D = 128
DE = 16
DT = 100
EMB = 128
H = 2
B = 4096
K = 20
KP = 24
DH = 114
P = 128
DQP = H * P

NC = 2
NS = 16
NW = NC * NS
QROWS_W = B // NW
NROWS_W = B * KP // NW
CHUNK = 128
NCH = NROWS_W // CHUNK
NBUF = 6
GA = 4


PASSES = NCH // NBUF


@functools.lru_cache(maxsize=None)
def _make_sc_gather():
    mesh = plsc.VectorSubcoreMesh(core_axis_name="c", subcore_axis_name="s")
    return functools.partial(
        pl.kernel,
        mesh=mesh,
        out_type=(jax.ShapeDtypeStruct((B, D), jnp.float32),
                  jax.ShapeDtypeStruct((B * KP, D // 2), jnp.int32)),
        scratch_types=[
            pltpu.VMEM_SHARED((NS, CHUNK + NROWS_W), jnp.int32),
            pltpu.SMEM((2, CHUNK), jnp.int32),
            pltpu.VMEM((QROWS_W, D), jnp.float32),
            pltpu.VMEM((NBUF, CHUNK, D // 2), jnp.int32),
        ] + [pltpu.SemaphoreType.DMA] * (2 * NBUF + 2),
    )(_sc_gather_body)


def _sc_gather(table, table_bf, ids):
    return _make_sc_gather()(table, table_bf, ids)


def _sc_gather_body(t32, tbf, idx_hbm, out_q, out_n, sp_idx, idx_s, qbuf, nbuf,
                    *sems):
    semg, semw = sems[:NBUF], sems[NBUF:2 * NBUF]
    qg, qw = sems[2 * NBUF], sems[2 * NBUF + 1]
    wid = lax.axis_index("s") * NC + lax.axis_index("c")
    nbase = wid * NROWS_W
    qslice = pl.ds(wid * QROWS_W, QROWS_W)

    sid = lax.axis_index("s")
    pltpu.sync_copy(idx_hbm.at[qslice], sp_idx.at[sid, pl.ds(0, CHUNK)])
    pltpu.sync_copy(idx_hbm.at[pl.ds(B + nbase, NROWS_W)],
                    sp_idx.at[sid, pl.ds(CHUNK, NROWS_W)])

    def fire_rows(slot, buf, sem, src=None):
        tab = tbf if src is None else src
        for j in range(CHUNK):
            i = idx_s[slot, j]
            pltpu.async_copy(tab.at[pl.ds(i, 1)], buf.at[pl.ds(j, 1)], sem)

    def stage_idx(c, slot):
        pltpu.sync_copy(sp_idx.at[sid, pl.ds(c * CHUNK, CHUNK)],
                        idx_s.at[slot])

    def drain_g(u):
        pltpu.make_async_copy(tbf.at[pl.ds(0, CHUNK)], nbuf.at[u],
                              semg[u]).wait()

    def fire_w(c, u):
        pltpu.async_copy(nbuf.at[u],
                         out_n.at[pl.ds(nbase + c * CHUNK, CHUNK)],
                         semw[u])

    def wait_w(c, u):
        pltpu.make_async_copy(nbuf.at[u],
                              out_n.at[pl.ds(nbase + c * CHUNK, CHUNK)],
                              semw[u]).wait()

    stage_idx(0, 0)
    fire_rows(0, qbuf, qg, src=t32)

    stage_idx(1, 0)
    for k in range(NBUF):
        if k + 1 < NCH:
            stage_idx(k + 2, (k + 1) % 2)
        fire_rows(k % 2, nbuf.at[k], semg[k])
        if k >= 1:
            drain_g(k - 1)
            fire_w(k - 1, k - 1)

    def ring_pass(p, carry):
        for k in range(NBUF):
            c = NBUF * p + k
            wait_w(c - NBUF, k)

            @pl.when(c + 1 < NCH)
            def _():
                pltpu.sync_copy(
                    sp_idx.at[sid, pl.ds((c + 2) * CHUNK, CHUNK)],
                    idx_s.at[(k + 1) % 2])

            fire_rows(k % 2, nbuf.at[k], semg[k])
            up = (k - 1) % NBUF
            drain_g(up)
            fire_w(c - 1, up)
        return carry

    lax.fori_loop(1, PASSES, ring_pass, 0)

    drain_g(NBUF - 1)
    fire_w(NCH - 1, NBUF - 1)

    pltpu.make_async_copy(t32.at[pl.ds(0, QROWS_W)], qbuf, qg).wait()
    pltpu.async_copy(qbuf, out_q.at[qslice], qw)
    for k in range(NBUF):
        wait_w(NCH - NBUF + k, k)
    pltpu.make_async_copy(qbuf, out_q.at[qslice], qw).wait()


BB = 128
GRID = B // BB
_INV_SQRT_DH = 1.0 / math.sqrt(DH)
_TWO_PI = 2.0 * math.pi
_INV_2PI = 1.0 / _TWO_PI
_CC = (9.99999211e-01, -4.99994213e-01, 4.16597776e-02,
       -1.38587892e-03, 2.42029321e-05, -2.19729219e-07)


def _cos_poly(x):
    r = x - _TWO_PI * jnp.floor(x * _INV_2PI + 0.5)
    y = r * r
    acc = jnp.float32(_CC[5])
    for c in (_CC[4], _CC[3], _CC[2], _CC[1], _CC[0]):
        acc = acc * y + c
    return acc


def _tc_body(nt_ref, nbt_ref, xg_ref, ng_ref, ef_ref, tw_ref, tb_ref,
             wqx_ref, wqt_ref, wkvx_ref, wkvet_ref,
             wo_ref, w1a_ref, w1b_ref, b1_ref, w2_ref, b2_ref, out_ref):
    f32 = jnp.float32
    x = xg_ref[...]
    n = ng_ref[...]
    ef = ef_ref[...]
    dt3 = (nt_ref[...].reshape(BB, 1, 1)
           - nbt_ref[...].reshape(BB, KP, 1))
    tw = tw_ref[...].reshape(1, 1, DT)
    tb = tb_ref[...].reshape(1, 1, DT)
    kt = _cos_poly(dt3 * tw + tb)
    et = jnp.concatenate([ef, kt.reshape(BB * KP, DT)], axis=-1)

    kv = (jnp.dot(n, wkvx_ref[...], preferred_element_type=f32)
          + jnp.dot(et, wkvet_ref[...], preferred_element_type=f32))

    qc = _cos_poly(tb_ref[...])
    q = (jnp.dot(x, wqx_ref[...], preferred_element_type=f32)
         + jnp.dot(qc, wqt_ref[...], preferred_element_type=f32))

    kv3 = kv.reshape(BB, KP, 2 * DQP)
    kmask3 = lax.broadcasted_iota(jnp.int32, (BB, KP, 1), 1) < K

    outs = []
    for h in range(H):
        qh = q[:, h * P:(h + 1) * P]
        kh = kv3[:, :, h * P:(h + 1) * P]
        vh = kv3[:, :, DQP + h * P:DQP + (h + 1) * P]
        s3 = (jnp.sum(kh * qh[:, None, :], axis=-1, keepdims=True)
              * _INV_SQRT_DH)
        s3 = jnp.where(kmask3, s3, -1e30)
        m = jnp.max(s3, axis=1, keepdims=True)
        e3 = jnp.exp(s3 - m)
        a3 = e3 / jnp.sum(e3, axis=1, keepdims=True)
        outs.append(jnp.sum(a3 * vh, axis=1))

    out = jnp.concatenate(outs, axis=-1)
    ao = jnp.dot(out, wo_ref[...], preferred_element_type=f32)
    h1 = jax.nn.relu(jnp.dot(ao, w1a_ref[...], preferred_element_type=f32)
                     + jnp.dot(x, w1b_ref[...], preferred_element_type=f32)
                     + b1_ref[...])
    out_ref[...] = (jnp.dot(h1, w2_ref[...], preferred_element_type=f32)
                    + b2_ref[...])


def _pad_cols(w):
    return jnp.concatenate(
        [jnp.pad(w[:, :DH], ((0, 0), (0, P - DH))),
         jnp.pad(w[:, DH:], ((0, 0), (0, P - DH)))], axis=1)


def kernel(node_feats, node_ids, node_times, nbr_ids, nbr_times, edge_feats,
           time_w, time_b, Wq, Wk, Wv, Wo, W1, b1, W2, b2):
    ids_p = jnp.pad(nbr_ids.astype(jnp.int32), ((0, 0), (0, KP - K)))
    all_ids = jnp.concatenate(
        [node_ids.astype(jnp.int32), ids_p.reshape(-1)])
    ef_p = jnp.pad(edge_feats,
                   ((0, 0), (0, KP - K), (0, 0))).reshape(B * KP, DE)
    nbt_p = jnp.pad(nbr_times, ((0, 0), (0, KP - K))).reshape(B * KP, 1)
    nt2 = node_times.reshape(B, 1)

    wq_p = _pad_cols(Wq)
    wqx, wqt = wq_p[:D], wq_p[D:]
    wkv = jnp.concatenate([_pad_cols(Wk), _pad_cols(Wv)], axis=1)
    wkvx, wkvet = wkv[:D].astype(jnp.bfloat16), wkv[D:]
    wo_p = jnp.concatenate(
        [jnp.pad(Wo[:DH], ((0, P - DH), (0, 0))),
         jnp.pad(Wo[DH:], ((0, P - DH), (0, 0)))], axis=0)
    w1a, w1b = W1[:D + DT], W1[D + DT:]
    b1r = b1.reshape(1, EMB)
    b2r = b2.reshape(1, EMB)
    twr = time_w.reshape(1, DT)
    tbr = time_b.reshape(1, DT)

    table_bf = jax.lax.bitcast_convert_type(
        node_feats.astype(jnp.bfloat16).reshape(N, D // 2, 2), jnp.int32)
    xg, ng32 = _sc_gather(node_feats, table_bf, all_ids)
    ngf = jax.lax.bitcast_convert_type(ng32, jnp.bfloat16).reshape(B * KP, D)

    full = lambda shape: pl.BlockSpec(shape, lambda i, s=shape: tuple(0 for _ in s))
    grid_spec = pl.GridSpec(
        grid=(GRID,),
        in_specs=[
            pl.BlockSpec((BB, 1), lambda i: (i, 0)),
            pl.BlockSpec((BB * KP, 1), lambda i: (i, 0)),
            pl.BlockSpec((BB, D), lambda i: (i, 0)),
            pl.BlockSpec((BB * KP, D), lambda i: (i, 0)),
            pl.BlockSpec((BB * KP, DE), lambda i: (i, 0)),
            full((1, DT)), full((1, DT)),
            full((D, DQP)), full((DT, DQP)),
            full((D, 2 * DQP)), full((DE + DT, 2 * DQP)),
            full((DQP, D + DT)),
            full((D + DT, EMB)), full((D, EMB)), full((1, EMB)),
            full((EMB, EMB)), full((1, EMB)),
        ],
        out_specs=pl.BlockSpec((BB, EMB), lambda i: (i, 0)),
    )
    h = pl.pallas_call(
        _tc_body,
        grid_spec=grid_spec,
        out_shape=jax.ShapeDtypeStruct((B, EMB), jnp.float32),
    )(nt2, nbt_p, xg, ngf, ef_p, twr, tbr,
      wqx, wqt, wkvx, wkvet, wo_p, w1a, w1b, b1r, W2, b2r)
    return h


# speedup vs baseline: 1.5171x; 1.5171x over previous
import functools
import math

import jax
import jax.numpy as jnp
from jax import lax
from jax.experimental import pallas as pl
from jax.experimental.pallas import tpu as pltpu
from jax.experimental.pallas import tpu_sc as plsc

N = 50000
D = 128
DE = 16
DT = 100
EMB = 128
H = 2
B = 4096
K = 20
KP = 24
DH = 114
P = 128
DQP = H * P

NC = 2
NS = 16
NW = NC * NS
QROWS_W = B // NW
NROWS_W = B * KP // NW
CHUNK = 128
NCH = NROWS_W // CHUNK
NBUF = 6
GA = 4


@functools.lru_cache(maxsize=None)
def _make_sc_gather():
    mesh = plsc.VectorSubcoreMesh(core_axis_name="c", subcore_axis_name="s")
    return functools.partial(
        pl.kernel,
        mesh=mesh,
        out_type=(jax.ShapeDtypeStruct((B, D), jnp.float32),
                  jax.ShapeDtypeStruct((B * KP, D), jnp.float32)),
        scratch_types=[
            pltpu.VMEM((CHUNK + NROWS_W,), jnp.int32),
            pltpu.VMEM((QROWS_W, D), jnp.float32),
            pltpu.VMEM((NBUF, CHUNK, D), jnp.float32),
        ] + [pltpu.SemaphoreType.DMA] * (2 * NBUF + 2),
    )(_sc_gather_body)


def _sc_gather(table, ids):
    return _make_sc_gather()(table, ids)


def _sc_gather_body(t32, idx_hbm, out_q, out_n, idx_all, qbuf, nbuf,
                    *sems):
    semg, semw = sems[:NBUF], sems[NBUF:2 * NBUF]
    qg, qw = sems[2 * NBUF], sems[2 * NBUF + 1]
    wid = lax.axis_index("s") * NC + lax.axis_index("c")
    nbase = wid * NROWS_W
    qslice = pl.ds(wid * QROWS_W, QROWS_W)

    pltpu.sync_copy(idx_hbm.at[qslice], idx_all.at[pl.ds(0, CHUNK)])
    pltpu.sync_copy(idx_hbm.at[pl.ds(B + nbase, NROWS_W)],
                    idx_all.at[pl.ds(CHUNK, NROWS_W)])
    idxq = idx_all.at[pl.ds(0, CHUNK)]

    def idxn(c):
        return idx_all.at[pl.ds((c + 1) * CHUNK, CHUNK)]

    def fire_g(c):
        pltpu.async_copy(t32.at[idxn(c)], nbuf.at[c % NBUF], semg[c % NBUF])

    def wait_g(c):
        pltpu.make_async_copy(t32.at[idxn(c)], nbuf.at[c % NBUF],
                              semg[c % NBUF]).wait()

    def fire_w(c):
        pltpu.async_copy(nbuf.at[c % NBUF],
                         out_n.at[pl.ds(nbase + c * CHUNK, CHUNK)],
                         semw[c % NBUF])

    def wait_w(c):
        pltpu.make_async_copy(nbuf.at[c % NBUF],
                              out_n.at[pl.ds(nbase + c * CHUNK, CHUNK)],
                              semw[c % NBUF]).wait()

    pltpu.async_copy(t32.at[idxq], qbuf, qg)
    for c in range(GA):
        fire_g(c)
    for c in range(NCH):
        nc = c + GA
        if nc < NCH:
            if nc >= NBUF:
                wait_w(nc - NBUF)
            fire_g(nc)
        wait_g(c)
        fire_w(c)
    pltpu.make_async_copy(t32.at[idxq], qbuf, qg).wait()
    pltpu.async_copy(qbuf, out_q.at[qslice], qw)
    for c in range(NCH - NBUF, NCH):
        wait_w(c)
    pltpu.make_async_copy(qbuf, out_q.at[qslice], qw).wait()


BB = 128
GRID = B // BB
_INV_SQRT_DH = 1.0 / math.sqrt(DH)
_TWO_PI = 2.0 * math.pi
_INV_2PI = 1.0 / _TWO_PI
_CC = (9.99999211e-01, -4.99994213e-01, 4.16597776e-02,
       -1.38587892e-03, 2.42029321e-05, -2.19729219e-07)


def _cos_poly(x):
    r = x - _TWO_PI * jnp.floor(x * _INV_2PI + 0.5)
    y = r * r
    acc = jnp.float32(_CC[5])
    for c in (_CC[4], _CC[3], _CC[2], _CC[1], _CC[0]):
        acc = acc * y + c
    return acc


def _tc_body(nt_ref, nbt_ref, xg_ref, ng_ref, ef_ref, tw_ref, tb_ref,
             wqx_ref, wqt_ref, wkvx_ref, wkvet_ref,
             wo_ref, w1a_ref, w1b_ref, b1_ref, w2_ref, b2_ref, out_ref):
    f32 = jnp.float32
    x = xg_ref[...]
    n = ng_ref[...].astype(jnp.bfloat16)
    ef = ef_ref[...]
    dt3 = (nt_ref[...].reshape(BB, 1, 1)
           - nbt_ref[...].reshape(BB, KP, 1))
    tw = tw_ref[...].reshape(1, 1, DT)
    tb = tb_ref[...].reshape(1, 1, DT)
    kt = _cos_poly(dt3 * tw + tb)
    et = jnp.concatenate([ef, kt.reshape(BB * KP, DT)], axis=-1)

    kv = (jnp.dot(n, wkvx_ref[...], preferred_element_type=f32)
          + jnp.dot(et, wkvet_ref[...], preferred_element_type=f32))

    qc = _cos_poly(tb_ref[...])
    q = (jnp.dot(x, wqx_ref[...], preferred_element_type=f32)
         + jnp.dot(qc, wqt_ref[...], preferred_element_type=f32))

    kv3 = kv.reshape(BB, KP, 2 * DQP)
    kmask3 = lax.broadcasted_iota(jnp.int32, (BB, KP, 1), 1) < K

    outs = []
    for h in range(H):
        qh = q[:, h * P:(h + 1) * P]
        kh = kv3[:, :, h * P:(h + 1) * P]
        vh = kv3[:, :, DQP + h * P:DQP + (h + 1) * P]
        s3 = (jnp.sum(kh * qh[:, None, :], axis=-1, keepdims=True)
              * _INV_SQRT_DH)
        s3 = jnp.where(kmask3, s3, -1e30)
        m = jnp.max(s3, axis=1, keepdims=True)
        e3 = jnp.exp(s3 - m)
        a3 = e3 / jnp.sum(e3, axis=1, keepdims=True)
        outs.append(jnp.sum(a3 * vh, axis=1))

    out = jnp.concatenate(outs, axis=-1)
    ao = jnp.dot(out, wo_ref[...], preferred_element_type=f32)
    h1 = jax.nn.relu(jnp.dot(ao, w1a_ref[...], preferred_element_type=f32)
                     + jnp.dot(x, w1b_ref[...], preferred_element_type=f32)
                     + b1_ref[...])
    out_ref[...] = (jnp.dot(h1, w2_ref[...], preferred_element_type=f32)
                    + b2_ref[...])


def _pad_cols(w):
    return jnp.concatenate(
        [jnp.pad(w[:, :DH], ((0, 0), (0, P - DH))),
         jnp.pad(w[:, DH:], ((0, 0), (0, P - DH)))], axis=1)


def kernel(node_feats, node_ids, node_times, nbr_ids, nbr_times, edge_feats,
           time_w, time_b, Wq, Wk, Wv, Wo, W1, b1, W2, b2):
    ids_p = jnp.pad(nbr_ids.astype(jnp.int32), ((0, 0), (0, KP - K)))
    all_ids = jnp.concatenate(
        [node_ids.astype(jnp.int32), ids_p.reshape(-1)])
    ef_p = jnp.pad(edge_feats,
                   ((0, 0), (0, KP - K), (0, 0))).reshape(B * KP, DE)
    nbt_p = jnp.pad(nbr_times, ((0, 0), (0, KP - K))).reshape(B * KP, 1)
    nt2 = node_times.reshape(B, 1)

    wq_p = _pad_cols(Wq)
    wqx, wqt = wq_p[:D], wq_p[D:]
    wkv = jnp.concatenate([_pad_cols(Wk), _pad_cols(Wv)], axis=1)
    wkvx, wkvet = wkv[:D].astype(jnp.bfloat16), wkv[D:]
    wo_p = jnp.concatenate(
        [jnp.pad(Wo[:DH], ((0, P - DH), (0, 0))),
         jnp.pad(Wo[DH:], ((0, P - DH), (0, 0)))], axis=0)
    w1a, w1b = W1[:D + DT], W1[D + DT:]
    b1r = b1.reshape(1, EMB)
    b2r = b2.reshape(1, EMB)
    twr = time_w.reshape(1, DT)
    tbr = time_b.reshape(1, DT)

    xg, ngf = _sc_gather(node_feats, all_ids)

    full = lambda shape: pl.BlockSpec(shape, lambda i, s=shape: tuple(0 for _ in s))
    grid_spec = pl.GridSpec(
        grid=(GRID,),
        in_specs=[
            pl.BlockSpec((BB, 1), lambda i: (i, 0)),
            pl.BlockSpec((BB * KP, 1), lambda i: (i, 0)),
            pl.BlockSpec((BB, D), lambda i: (i, 0)),
            pl.BlockSpec((BB * KP, D), lambda i: (i, 0)),
            pl.BlockSpec((BB * KP, DE), lambda i: (i, 0)),
            full((1, DT)), full((1, DT)),
            full((D, DQP)), full((DT, DQP)),
            full((D, 2 * DQP)), full((DE + DT, 2 * DQP)),
            full((DQP, D + DT)),
            full((D + DT, EMB)), full((D, EMB)), full((1, EMB)),
            full((EMB, EMB)), full((1, EMB)),
        ],
        out_specs=pl.BlockSpec((BB, EMB), lambda i: (i, 0)),
    )
    h = pl.pallas_call(
        _tc_body,
        grid_spec=grid_spec,
        out_shape=jax.ShapeDtypeStruct((B, EMB), jnp.float32),
    )(nt2, nbt_p, xg, ngf, ef_p, twr, tbr,
      wqx, wqt, wkvx, wkvet, wo_p, w1a, w1b, b1r, W2, b2r)
    return h
